# baseline (speedup 1.0000x reference)
import jax
import jax.numpy as jnp
from jax import lax
from jax.experimental import pallas as pl
from jax.experimental.pallas import tpu as pltpu
from jax.experimental.pallas import tpu_sc as plsc

N_KV_HEADS = 8
HEAD_DIM = 128
MAX_SEQ_LEN = 8192
Q_LEN = 16
NROWS = N_KV_HEADS * Q_LEN


def _tc_body(pos_ref, kc_ref, kval_ref, ko_ref):
    ko_ref[...] = kc_ref[...]
    for i in range(Q_LEN):
        p = pos_ref[i]
        ko_ref[0, pl.ds(p, 1), :] = kval_ref[0, pl.ds(i, 1), :]


def _tc_update(kc, kv, pos):
    cache_spec = pl.BlockSpec((1, MAX_SEQ_LEN, HEAD_DIM), lambda h, pos_ref: (h, 0, 0))
    val_spec = pl.BlockSpec((1, Q_LEN, HEAD_DIM), lambda h, pos_ref: (h, 0, 0))
    grid_spec = pltpu.PrefetchScalarGridSpec(
        num_scalar_prefetch=1,
        grid=(N_KV_HEADS,),
        in_specs=[cache_spec, val_spec],
        out_specs=[cache_spec],
    )
    return pl.pallas_call(
        _tc_body,
        grid_spec=grid_spec,
        out_shape=[jax.ShapeDtypeStruct(kc.shape, kc.dtype)],
    )(pos, kc, kv)[0]


def _sc_body(v_ref, pos_hbm, vval_hbm, pos_v, idx_v, rows_v, sem):
    wid = lax.axis_index("s") * 2 + lax.axis_index("c")

    @pl.when(wid == 0)
    def _():
        pltpu.sync_copy(pos_hbm, pos_v)
        lanes = lax.iota(jnp.int32, Q_LEN)
        pos_vec = pos_v[...]
        w = lanes
        for shift in range(1, Q_LEN):
            perm = (lanes + shift) & (Q_LEN - 1)
            p_sh = lax.gather(
                pos_vec, perm[:, None],
                lax.GatherDimensionNumbers(
                    offset_dims=(), collapsed_slice_dims=(0,),
                    start_index_map=(0,)),
                slice_sizes=(1,),
                mode=lax.GatherScatterMode.PROMISE_IN_BOUNDS)
            w = jnp.where(p_sh == pos_vec, jnp.maximum(w, perm), w)
        for h in range(N_KV_HEADS):
            idx_v[pl.ds(h * Q_LEN, Q_LEN)] = w + h * Q_LEN
        pltpu.async_copy(vval_hbm.at[idx_v], rows_v, sem).wait()
        for h in range(N_KV_HEADS):
            idx_v[pl.ds(h * Q_LEN, Q_LEN)] = pos_vec + h * MAX_SEQ_LEN
        pltpu.async_copy(rows_v, v_ref.at[idx_v], sem).wait()


_sc_update = pl.kernel(
    _sc_body,
    out_type=(),
    mesh=plsc.VectorSubcoreMesh(core_axis_name="c", subcore_axis_name="s"),
    scratch_types=[
        pltpu.VMEM((Q_LEN,), jnp.int32),
        pltpu.VMEM((NROWS,), jnp.int32),
        pltpu.VMEM((NROWS, HEAD_DIM), jnp.float32),
        pltpu.SemaphoreType.DMA,
    ],
)


def kernel(k_cache, v_cache, input_pos, k_val, v_val):
    kc = k_cache.reshape(N_KV_HEADS, MAX_SEQ_LEN, HEAD_DIM)
    vc = v_cache.reshape(N_KV_HEADS * MAX_SEQ_LEN, HEAD_DIM)
    kv = k_val.reshape(N_KV_HEADS, Q_LEN, HEAD_DIM)
    vv = v_val.reshape(NROWS, HEAD_DIM)
    pos = input_pos.astype(jnp.int32)

    ko = _tc_update(kc, kv, pos)

    v_ref = jax.new_ref(vc)
    _sc_update(v_ref, pos, vv)
    vo = v_ref[...]

    return (ko.reshape(k_cache.shape), vo.reshape(v_cache.shape))


# speedup vs baseline: 1.0004x; 1.0004x over previous
import jax
import jax.numpy as jnp
from jax import lax
from jax.experimental import pallas as pl
from jax.experimental.pallas import tpu as pltpu
from jax.experimental.pallas import tpu_sc as plsc

N_KV_HEADS = 8
HEAD_DIM = 128
MAX_SEQ_LEN = 8192
Q_LEN = 16
NROWS = N_KV_HEADS * Q_LEN


def _tc_copy_body(kc_ref, vc_ref, ko_ref, vo_ref):
    ko_ref[...] = kc_ref[...]
    vo_ref[...] = vc_ref[...]


def _tc_copy(kc, vc):
    cache_spec = pl.BlockSpec((1, MAX_SEQ_LEN, HEAD_DIM), lambda h: (h, 0, 0))
    return pl.pallas_call(
        _tc_copy_body,
        grid=(N_KV_HEADS,),
        in_specs=[cache_spec, cache_spec],
        out_specs=[cache_spec, cache_spec],
        out_shape=[
            jax.ShapeDtypeStruct(kc.shape, kc.dtype),
            jax.ShapeDtypeStruct(vc.shape, vc.dtype),
        ],
    )(kc, vc)


def _sc_body(k_ref, v_ref, pos_hbm, kval_hbm, vval_hbm,
             pos_v, idx_v, rows_v, sem):
    wid = lax.axis_index("s") * 2 + lax.axis_index("c")

    def do_cache(cache_ref, val_hbm):
        pltpu.sync_copy(pos_hbm, pos_v)
        lanes = lax.iota(jnp.int32, Q_LEN)
        pos_vec = pos_v[...]
        w = lanes
        for shift in range(1, Q_LEN):
            perm = (lanes + shift) & (Q_LEN - 1)
            p_sh = lax.gather(
                pos_vec, perm[:, None],
                lax.GatherDimensionNumbers(
                    offset_dims=(), collapsed_slice_dims=(0,),
                    start_index_map=(0,)),
                slice_sizes=(1,),
                mode=lax.GatherScatterMode.PROMISE_IN_BOUNDS)
            w = jnp.where(p_sh == pos_vec, jnp.maximum(w, perm), w)
        for h in range(N_KV_HEADS):
            idx_v[pl.ds(h * Q_LEN, Q_LEN)] = w + h * Q_LEN
        pltpu.async_copy(val_hbm.at[idx_v], rows_v, sem).wait()
        for h in range(N_KV_HEADS):
            idx_v[pl.ds(h * Q_LEN, Q_LEN)] = pos_vec + h * MAX_SEQ_LEN
        pltpu.async_copy(rows_v, cache_ref.at[idx_v], sem).wait()

    @pl.when(wid == 0)
    def _():
        do_cache(k_ref, kval_hbm)

    @pl.when(wid == 1)
    def _():
        do_cache(v_ref, vval_hbm)


_sc_update = pl.kernel(
    _sc_body,
    out_type=(),
    mesh=plsc.VectorSubcoreMesh(core_axis_name="c", subcore_axis_name="s"),
    scratch_types=[
        pltpu.VMEM((Q_LEN,), jnp.int32),
        pltpu.VMEM((NROWS,), jnp.int32),
        pltpu.VMEM((NROWS, HEAD_DIM), jnp.float32),
        pltpu.SemaphoreType.DMA,
    ],
)


def kernel(k_cache, v_cache, input_pos, k_val, v_val):
    kc = k_cache.reshape(N_KV_HEADS, MAX_SEQ_LEN, HEAD_DIM)
    vc = v_cache.reshape(N_KV_HEADS, MAX_SEQ_LEN, HEAD_DIM)
    kv = k_val.reshape(NROWS, HEAD_DIM)
    vv = v_val.reshape(NROWS, HEAD_DIM)
    pos = input_pos.astype(jnp.int32)

    ko_raw, vo_raw = _tc_copy(kc, vc)

    k_ref = jax.new_ref(ko_raw.reshape(N_KV_HEADS * MAX_SEQ_LEN, HEAD_DIM))
    v_ref = jax.new_ref(vo_raw.reshape(N_KV_HEADS * MAX_SEQ_LEN, HEAD_DIM))
    _sc_update(k_ref, v_ref, pos, kv, vv)
    ko = k_ref[...]
    vo = v_ref[...]
    return (ko.reshape(k_cache.shape), vo.reshape(v_cache.shape))


# speedup vs baseline: 1.4248x; 1.4242x over previous
import jax
import jax.numpy as jnp
from jax.experimental import pallas as pl
from jax.experimental.pallas import tpu as pltpu

N_KV_HEADS = 8
HEAD_DIM = 128
MAX_SEQ_LEN = 8192
Q_LEN = 16

HB = 1
NHB = N_KV_HEADS // HB


def _update_body(pos_ref, kc_ref, vc_ref, kval_ref, vval_ref, ko_ref, vo_ref):
    ko_ref[...] = kc_ref[...]
    vo_ref[...] = vc_ref[...]
    for i in range(Q_LEN):
        p = pos_ref[i]
        for lh in range(HB):
            ko_ref[lh, pl.ds(p, 1), :] = kval_ref[lh, pl.ds(i, 1), :]
            vo_ref[lh, pl.ds(p, 1), :] = vval_ref[lh, pl.ds(i, 1), :]


def kernel(k_cache, v_cache, input_pos, k_val, v_val):
    kc = k_cache.reshape(N_KV_HEADS, MAX_SEQ_LEN, HEAD_DIM)
    vc = v_cache.reshape(N_KV_HEADS, MAX_SEQ_LEN, HEAD_DIM)
    kv = k_val.reshape(N_KV_HEADS, Q_LEN, HEAD_DIM)
    vv = v_val.reshape(N_KV_HEADS, Q_LEN, HEAD_DIM)
    pos = input_pos.astype(jnp.int32)

    cache_spec = pl.BlockSpec(
        (HB, MAX_SEQ_LEN, HEAD_DIM), lambda h, pos_ref: (h, 0, 0))
    val_spec = pl.BlockSpec((HB, Q_LEN, HEAD_DIM), lambda h, pos_ref: (h, 0, 0))

    grid_spec = pltpu.PrefetchScalarGridSpec(
        num_scalar_prefetch=1,
        grid=(NHB,),
        in_specs=[cache_spec, cache_spec, val_spec, val_spec],
        out_specs=[cache_spec, cache_spec],
    )

    ko, vo = pl.pallas_call(
        _update_body,
        grid_spec=grid_spec,
        out_shape=[
            jax.ShapeDtypeStruct(kc.shape, kc.dtype),
            jax.ShapeDtypeStruct(vc.shape, vc.dtype),
        ],
        compiler_params=pltpu.CompilerParams(
            vmem_limit_bytes=100 * 1024 * 1024,
        ),
    )(pos, kc, vc, kv, vv)

    return (ko.reshape(k_cache.shape), vo.reshape(v_cache.shape))
